# baseline (speedup 1.0000x reference)
import functools

import jax
import jax.numpy as jnp
from jax import lax
from jax.experimental import pallas as pl
from jax.experimental.pallas import tpu as pltpu
from jax.experimental.pallas import tpu_sc as plsc

N = 10000
E = 320000
F = 128
NC = 2
NS = 16
NW = NC * NS
EPW = E // NW
B = 80
KCH = EPW // B
NPT = N // NS
NBUF = 3
PF = 1
WB = NPT // B
WBR = NPT - WB * B

_mesh = plsc.VectorSubcoreMesh(core_axis_name="c", subcore_axis_name="s")
_sc_params = pltpu.CompilerParams(use_tc_tiling_on_sc=False)


@functools.partial(
    pl.kernel,
    out_type=jax.ShapeDtypeStruct((NC, N), jnp.float32),
    mesh=_mesh,
    compiler_params=_sc_params,
    scratch_types=[
        pltpu.VMEM((KCH, B), jnp.int32),
        pltpu.VMEM((B,), jnp.float32),
        pltpu.VMEM((N,), jnp.float32),
        pltpu.VMEM_SHARED((N,), jnp.float32),
        pltpu.SemaphoreType.DMA,
    ],
)
def _sc_degree(ei_hbm, ones_hbm, zeros_hbm, out_hbm, cidx_v, ones_v, bounce_v,
               deg_sh, asem):
    c = lax.axis_index("c")
    s = lax.axis_index("s")
    w = c * NS + s

    @pl.when(s == 0)
    def _zero():
        pltpu.sync_copy(zeros_hbm, bounce_v)
        pltpu.sync_copy(bounce_v, deg_sh)

    pltpu.sync_copy(ones_hbm, ones_v)
    pltpu.sync_copy(ei_hbm.at[1, pl.ds(w * KCH, KCH)], cidx_v)
    plsc.subcore_barrier()

    def body(k, carry):
        pltpu.async_copy(ones_v, deg_sh.at[cidx_v.at[k]], asem, add=True)
        return carry

    lax.fori_loop(0, KCH, body, 0)
    pltpu.make_async_copy(zeros_hbm, bounce_v, asem).wait()
    plsc.subcore_barrier()

    @pl.when(s == 0)
    def _writeback():
        pltpu.sync_copy(deg_sh, bounce_v)
        pltpu.sync_copy(bounce_v, out_hbm.at[c])


@functools.partial(
    pl.kernel,
    out_type=jax.ShapeDtypeStruct((NC, N, F), jnp.float32),
    mesh=_mesh,
    compiler_params=_sc_params,
    scratch_types=[
        pltpu.VMEM((KCH, B), jnp.int32),
        pltpu.VMEM((KCH, B), jnp.int32),
        [pltpu.VMEM((B, F), jnp.float32)] * NBUF,
        pltpu.VMEM_SHARED((N, F), jnp.float32),
        [pltpu.SemaphoreType.DMA] * NBUF,
        [pltpu.SemaphoreType.DMA] * NBUF,
    ],
)
def _sc_scatter(ei_hbm, g_hbm, zeros_hbm, out_hbm,
                ridx_v, cidx_v, rows_v, acc_sh, gsem, ssem):
    c = lax.axis_index("c")
    s = lax.axis_index("s")
    w = c * NS + s

    pltpu.sync_copy(zeros_hbm, rows_v[0])
    pltpu.sync_copy(ei_hbm.at[0, pl.ds(w * KCH, KCH)], ridx_v)
    pltpu.sync_copy(ei_hbm.at[1, pl.ds(w * KCH, KCH)], cidx_v)
    for j in range(WB):
        pltpu.async_copy(rows_v[0], acc_sh.at[pl.ds(s * NPT + j * B, B)],
                         ssem[0])
    if WBR:
        pltpu.async_copy(rows_v[0].at[pl.ds(0, WBR)],
                         acc_sh.at[pl.ds(s * NPT + WB * B, WBR)], ssem[0])
    for j in range(WB):
        pltpu.make_async_copy(rows_v[0],
                              acc_sh.at[pl.ds(s * NPT + j * B, B)],
                              ssem[0]).wait()
    if WBR:
        pltpu.make_async_copy(rows_v[0].at[pl.ds(0, WBR)],
                              acc_sh.at[pl.ds(s * NPT + WB * B, WBR)],
                              ssem[0]).wait()
    plsc.subcore_barrier()

    for i in range(PF):
        pltpu.async_copy(g_hbm.at[ridx_v.at[i]], rows_v[i], gsem[i])

    def body(k0, carry):
        for phase in range(NBUF):
            k = k0 * NBUF + phase
            bi = phase % NBUF

            @pl.when(k < KCH)
            def _work():
                pltpu.make_async_copy(
                    g_hbm.at[ridx_v.at[k]], rows_v[bi], gsem[bi]).wait()
                pltpu.async_copy(rows_v[bi], acc_sh.at[cidx_v.at[k]],
                                 ssem[bi], add=True)

            bj = (phase + PF) % NBUF

            @pl.when(jnp.logical_and(k >= NBUF - PF, k + PF < KCH))
            def _drain_prev():
                pltpu.make_async_copy(
                    rows_v[bj], acc_sh.at[cidx_v.at[0]], ssem[bj]).wait()

            @pl.when(k + PF < KCH)
            def _prefetch():
                pltpu.async_copy(
                    g_hbm.at[ridx_v.at[k + PF]], rows_v[bj], gsem[bj])
        return carry

    lax.fori_loop(0, (KCH + NBUF - 1) // NBUF, body, 0)
    for bi in range(NBUF):
        pltpu.make_async_copy(rows_v[bi], acc_sh.at[cidx_v.at[0]],
                              ssem[bi]).wait()
    plsc.subcore_barrier()

    nwb = WB + (1 if WBR else 0)
    for j in range(nwb):
        bi = j % NBUF
        nrows = B if j < WB else WBR
        off = s * NPT + j * B
        src = rows_v[bi] if j < WB else rows_v[bi].at[pl.ds(0, WBR)]
        if j >= NBUF:
            poff = s * NPT + (j - NBUF) * B
            pltpu.make_async_copy(rows_v[bi], out_hbm.at[c, pl.ds(poff, B)],
                                  gsem[bi]).wait()
        pltpu.sync_copy(acc_sh.at[pl.ds(off, nrows)], src)
        pltpu.async_copy(src, out_hbm.at[c, pl.ds(off, nrows)], gsem[bi])
    for j in range(max(0, nwb - NBUF), nwb):
        bi = j % NBUF
        nrows = B if j < WB else WBR
        off = s * NPT + j * B
        src = rows_v[bi] if j < WB else rows_v[bi].at[pl.ds(0, WBR)]
        pltpu.make_async_copy(src, out_hbm.at[c, pl.ds(off, nrows)],
                              gsem[bi]).wait()


def _tc_linear_body(x_ref, w_ref, u_ref, h_ref):
    Wm = w_ref[...]
    uv = u_ref[...]
    v = jnp.dot(uv, Wm, preferred_element_type=jnp.float32)
    v = v / (jnp.sqrt(jnp.sum(v * v)) + 1e-12)
    wv = jnp.dot(v, Wm.T, preferred_element_type=jnp.float32)
    u2 = wv / (jnp.sqrt(jnp.sum(wv * wv)) + 1e-12)
    sigma = jnp.sum(u2 * wv)
    h_ref[...] = jnp.dot(x_ref[...], Wm,
                         preferred_element_type=jnp.float32) / sigma


def _tc_scale_body(h_ref, degp_ref, g_ref):
    deg = degp_ref[:, 0:1] + degp_ref[:, 1:2] + 1.0
    g_ref[...] = h_ref[...] * lax.rsqrt(deg)


def _tc_combine_body(acc_ref, g_ref, degp_ref, b_ref, a_ref, out_ref):
    total = acc_ref[0] + acc_ref[1] + g_ref[...]
    deg = degp_ref[:, 0:1] + degp_ref[:, 1:2] + 1.0
    o = total * lax.rsqrt(deg) + b_ref[...]
    a = a_ref[0, 0]
    out_ref[...] = jnp.where(o >= 0, o, a * o)


def kernel(x, edge_index, W, b, prelu_a, u):
    ei3 = edge_index.astype(jnp.int32).reshape(2, E // B, B)

    ones_b = jnp.ones((B,), jnp.float32)
    zeros_n = jnp.zeros((N,), jnp.float32)
    zeros_zr = jnp.zeros((B, F), jnp.float32)

    deg_p = _sc_degree(ei3, ones_b, zeros_n)
    degp_t = deg_p.T

    R = 2000
    grid = (N // R,)
    h = pl.pallas_call(
        _tc_linear_body,
        grid=grid,
        in_specs=[
            pl.BlockSpec((R, F), lambda i: (i, 0)),
            pl.BlockSpec((F, F), lambda i: (0, 0)),
            pl.BlockSpec((1, F), lambda i: (0, 0)),
        ],
        out_specs=pl.BlockSpec((R, F), lambda i: (i, 0)),
        out_shape=jax.ShapeDtypeStruct((N, F), jnp.float32),
    )(x, W, u.reshape(1, F))

    g = pl.pallas_call(
        _tc_scale_body,
        grid=grid,
        in_specs=[
            pl.BlockSpec((R, F), lambda i: (i, 0)),
            pl.BlockSpec((R, 2), lambda i: (i, 0)),
        ],
        out_specs=pl.BlockSpec((R, F), lambda i: (i, 0)),
        out_shape=jax.ShapeDtypeStruct((N, F), jnp.float32),
    )(h, degp_t)

    acc = _sc_scatter(ei3, g, zeros_zr)

    out = pl.pallas_call(
        _tc_combine_body,
        grid=grid,
        in_specs=[
            pl.BlockSpec((NC, R, F), lambda i: (0, i, 0)),
            pl.BlockSpec((R, F), lambda i: (i, 0)),
            pl.BlockSpec((R, 2), lambda i: (i, 0)),
            pl.BlockSpec((1, F), lambda i: (0, 0)),
            pl.BlockSpec((1, 1), lambda i: (0, 0)),
        ],
        out_specs=pl.BlockSpec((R, F), lambda i: (i, 0)),
        out_shape=jax.ShapeDtypeStruct((N, F), jnp.float32),
    )(acc, g, degp_t, b.reshape(1, F), prelu_a.reshape(1, 1))
    return out


# speedup vs baseline: 1.2681x; 1.2681x over previous
import functools

import jax
import jax.numpy as jnp
from jax import lax
from jax.experimental import pallas as pl
from jax.experimental.pallas import tpu as pltpu
from jax.experimental.pallas import tpu_sc as plsc

N = 10000
E = 320000
F = 128
NC = 2
NS = 16
NW = NC * NS
EPW = E // NW
B = 80
KCH = EPW // B
NPT = N // NS
HF = F // NC
EPT = E // NS
KCS = EPT // B
NBUF = 8
PF = 6
WB = NPT // B
WBR = NPT - WB * B

_mesh = plsc.VectorSubcoreMesh(core_axis_name="c", subcore_axis_name="s")
_sc_params = pltpu.CompilerParams(use_tc_tiling_on_sc=False)


@functools.partial(
    pl.kernel,
    out_type=jax.ShapeDtypeStruct((NC, N), jnp.float32),
    mesh=_mesh,
    compiler_params=_sc_params,
    scratch_types=[
        pltpu.VMEM((KCH, B), jnp.int32),
        pltpu.VMEM((B,), jnp.float32),
        pltpu.VMEM((N,), jnp.float32),
        pltpu.VMEM_SHARED((N,), jnp.float32),
        pltpu.SemaphoreType.DMA,
    ],
)
def _sc_degree(ei_hbm, ones_hbm, zeros_hbm, out_hbm, cidx_v, ones_v, bounce_v,
               deg_sh, asem):
    c = lax.axis_index("c")
    s = lax.axis_index("s")
    w = c * NS + s

    @pl.when(s == 0)
    def _zero():
        pltpu.sync_copy(zeros_hbm, bounce_v)
        pltpu.sync_copy(bounce_v, deg_sh)

    pltpu.sync_copy(ones_hbm, ones_v)
    pltpu.sync_copy(ei_hbm.at[1, pl.ds(w * KCH, KCH)], cidx_v)
    plsc.subcore_barrier()

    def body(k, carry):
        pltpu.async_copy(ones_v, deg_sh.at[cidx_v.at[k]], asem, add=True)
        return carry

    lax.fori_loop(0, KCH, body, 0)
    pltpu.make_async_copy(zeros_hbm, bounce_v, asem).wait()
    plsc.subcore_barrier()

    @pl.when(s == 0)
    def _writeback():
        pltpu.sync_copy(deg_sh, bounce_v)
        pltpu.sync_copy(bounce_v, out_hbm.at[c])


@functools.partial(
    pl.kernel,
    out_type=jax.ShapeDtypeStruct((NC, N, HF), jnp.float32),
    mesh=_mesh,
    compiler_params=_sc_params,
    scratch_types=[
        pltpu.VMEM((KCS, B), jnp.int32),
        pltpu.VMEM((KCS, B), jnp.int32),
        [pltpu.VMEM((B, HF), jnp.float32)] * NBUF,
        pltpu.VMEM_SHARED((N, HF), jnp.float32),
        [pltpu.SemaphoreType.DMA] * NBUF,
        [pltpu.SemaphoreType.DMA] * NBUF,
    ],
)
def _sc_scatter(ei_hbm, g_hbm, zeros_hbm, out_hbm,
                ridx_v, cidx_v, rows_v, acc_sh, gsem, ssem):
    c = lax.axis_index("c")
    s = lax.axis_index("s")

    pltpu.sync_copy(zeros_hbm, rows_v[0])
    pltpu.sync_copy(ei_hbm.at[0, pl.ds(s * KCS, KCS)], ridx_v)
    pltpu.sync_copy(ei_hbm.at[1, pl.ds(s * KCS, KCS)], cidx_v)
    for j in range(WB):
        pltpu.async_copy(rows_v[0], acc_sh.at[pl.ds(s * NPT + j * B, B)],
                         ssem[0])
    if WBR:
        pltpu.async_copy(rows_v[0].at[pl.ds(0, WBR)],
                         acc_sh.at[pl.ds(s * NPT + WB * B, WBR)], ssem[0])
    for j in range(WB):
        pltpu.make_async_copy(rows_v[0],
                              acc_sh.at[pl.ds(s * NPT + j * B, B)],
                              ssem[0]).wait()
    if WBR:
        pltpu.make_async_copy(rows_v[0].at[pl.ds(0, WBR)],
                              acc_sh.at[pl.ds(s * NPT + WB * B, WBR)],
                              ssem[0]).wait()
    plsc.subcore_barrier()

    gc = g_hbm.at[c]
    for i in range(PF):
        pltpu.async_copy(gc.at[ridx_v.at[i]], rows_v[i], gsem[i])

    def body(k0, carry):
        for phase in range(NBUF):
            k = k0 * NBUF + phase
            bi = phase % NBUF

            @pl.when(k < KCS)
            def _work():
                pltpu.make_async_copy(
                    gc.at[ridx_v.at[k]], rows_v[bi], gsem[bi]).wait()
                pltpu.async_copy(rows_v[bi], acc_sh.at[cidx_v.at[k]],
                                 ssem[bi], add=True)

            bj = (phase + PF) % NBUF

            @pl.when(jnp.logical_and(k >= NBUF - PF, k + PF < KCS))
            def _drain_prev():
                pltpu.make_async_copy(
                    rows_v[bj], acc_sh.at[cidx_v.at[0]], ssem[bj]).wait()

            @pl.when(k + PF < KCS)
            def _prefetch():
                pltpu.async_copy(
                    gc.at[ridx_v.at[k + PF]], rows_v[bj], gsem[bj])
        return carry

    lax.fori_loop(0, (KCS + NBUF - 1) // NBUF, body, 0)
    for bi in range(NBUF):
        pltpu.make_async_copy(rows_v[bi], acc_sh.at[cidx_v.at[0]],
                              ssem[bi]).wait()
    plsc.subcore_barrier()

    nwb = WB + (1 if WBR else 0)
    for j in range(nwb):
        bi = j % NBUF
        nrows = B if j < WB else WBR
        off = s * NPT + j * B
        src = rows_v[bi] if j < WB else rows_v[bi].at[pl.ds(0, WBR)]
        if j >= NBUF:
            poff = s * NPT + (j - NBUF) * B
            pltpu.make_async_copy(rows_v[bi], out_hbm.at[c, pl.ds(poff, B)],
                                  gsem[bi]).wait()
        pltpu.sync_copy(acc_sh.at[pl.ds(off, nrows)], src)
        pltpu.async_copy(src, out_hbm.at[c, pl.ds(off, nrows)], gsem[bi])
    for j in range(max(0, nwb - NBUF), nwb):
        bi = j % NBUF
        nrows = B if j < WB else WBR
        off = s * NPT + j * B
        src = rows_v[bi] if j < WB else rows_v[bi].at[pl.ds(0, WBR)]
        pltpu.make_async_copy(src, out_hbm.at[c, pl.ds(off, nrows)],
                              gsem[bi]).wait()


def _tc_linear_body(x_ref, w_ref, u_ref, h_ref):
    Wm = w_ref[...]
    uv = u_ref[...]
    v = jnp.dot(uv, Wm, preferred_element_type=jnp.float32)
    v = v / (jnp.sqrt(jnp.sum(v * v)) + 1e-12)
    wv = jnp.dot(v, Wm.T, preferred_element_type=jnp.float32)
    u2 = wv / (jnp.sqrt(jnp.sum(wv * wv)) + 1e-12)
    sigma = jnp.sum(u2 * wv)
    h_ref[...] = jnp.dot(x_ref[...], Wm,
                         preferred_element_type=jnp.float32) / sigma


def _tc_scale_body(h_ref, degp_ref, gs_ref):
    deg = degp_ref[:, 0:1] + degp_ref[:, 1:2] + 1.0
    g = h_ref[...] * lax.rsqrt(deg)
    gs_ref[0] = g[:, :HF]
    gs_ref[1] = g[:, HF:]


def _tc_combine_body(acc_ref, gs_ref, degp_ref, b_ref, a_ref, out_ref):
    total = jnp.concatenate(
        [acc_ref[0] + gs_ref[0], acc_ref[1] + gs_ref[1]], axis=1)
    deg = degp_ref[:, 0:1] + degp_ref[:, 1:2] + 1.0
    o = total * lax.rsqrt(deg) + b_ref[...]
    a = a_ref[0, 0]
    out_ref[...] = jnp.where(o >= 0, o, a * o)


def kernel(x, edge_index, W, b, prelu_a, u):
    ei3 = edge_index.astype(jnp.int32).reshape(2, E // B, B)

    ones_b = jnp.ones((B,), jnp.float32)
    zeros_n = jnp.zeros((N,), jnp.float32)
    zeros_zr = jnp.zeros((B, HF), jnp.float32)

    deg_p = _sc_degree(ei3, ones_b, zeros_n)
    degp_t = deg_p.T

    R = 2000
    grid = (N // R,)
    h = pl.pallas_call(
        _tc_linear_body,
        grid=grid,
        in_specs=[
            pl.BlockSpec((R, F), lambda i: (i, 0)),
            pl.BlockSpec((F, F), lambda i: (0, 0)),
            pl.BlockSpec((1, F), lambda i: (0, 0)),
        ],
        out_specs=pl.BlockSpec((R, F), lambda i: (i, 0)),
        out_shape=jax.ShapeDtypeStruct((N, F), jnp.float32),
    )(x, W, u.reshape(1, F))

    gs = pl.pallas_call(
        _tc_scale_body,
        grid=grid,
        in_specs=[
            pl.BlockSpec((R, F), lambda i: (i, 0)),
            pl.BlockSpec((R, 2), lambda i: (i, 0)),
        ],
        out_specs=pl.BlockSpec((NC, R, HF), lambda i: (0, i, 0)),
        out_shape=jax.ShapeDtypeStruct((NC, N, HF), jnp.float32),
    )(h, degp_t)

    acc = _sc_scatter(ei3, gs, zeros_zr)

    out = pl.pallas_call(
        _tc_combine_body,
        grid=grid,
        in_specs=[
            pl.BlockSpec((NC, R, HF), lambda i: (0, i, 0)),
            pl.BlockSpec((NC, R, HF), lambda i: (0, i, 0)),
            pl.BlockSpec((R, 2), lambda i: (i, 0)),
            pl.BlockSpec((1, F), lambda i: (0, 0)),
            pl.BlockSpec((1, 1), lambda i: (0, 0)),
        ],
        out_specs=pl.BlockSpec((R, F), lambda i: (i, 0)),
        out_shape=jax.ShapeDtypeStruct((N, F), jnp.float32),
    )(acc, gs, degp_t, b.reshape(1, F), prelu_a.reshape(1, 1))
    return out


# speedup vs baseline: 1.3714x; 1.0815x over previous
import functools

import jax
import jax.numpy as jnp
from jax import lax
from jax.experimental import pallas as pl
from jax.experimental.pallas import tpu as pltpu
from jax.experimental.pallas import tpu_sc as plsc

N = 10000
E = 320000
F = 128
NC = 2
NS = 16
NW = NC * NS
EPW = E // NW
B = 80
KCH = EPW // B
NPT = N // NS
NBUF = 3
PF = 2
WB = NPT // B
WBR = NPT - WB * B

_mesh = plsc.VectorSubcoreMesh(core_axis_name="c", subcore_axis_name="s")
_sc_params = pltpu.CompilerParams(use_tc_tiling_on_sc=False)


@functools.partial(
    pl.kernel,
    out_type=jax.ShapeDtypeStruct((NC, N), jnp.float32),
    mesh=_mesh,
    compiler_params=_sc_params,
    scratch_types=[
        pltpu.VMEM((KCH, B), jnp.int32),
        pltpu.VMEM((B,), jnp.float32),
        pltpu.VMEM((N,), jnp.float32),
        pltpu.VMEM_SHARED((N,), jnp.float32),
        pltpu.SemaphoreType.DMA,
    ],
)
def _sc_degree(ei_hbm, ones_hbm, zeros_hbm, out_hbm, cidx_v, ones_v, bounce_v,
               deg_sh, asem):
    c = lax.axis_index("c")
    s = lax.axis_index("s")
    w = c * NS + s

    @pl.when(s == 0)
    def _zero():
        pltpu.sync_copy(zeros_hbm, bounce_v)
        pltpu.sync_copy(bounce_v, deg_sh)

    pltpu.sync_copy(ones_hbm, ones_v)
    pltpu.sync_copy(ei_hbm.at[1, pl.ds(w * KCH, KCH)], cidx_v)
    plsc.subcore_barrier()

    def body(k, carry):
        pltpu.async_copy(ones_v, deg_sh.at[cidx_v.at[k]], asem, add=True)
        return carry

    lax.fori_loop(0, KCH, body, 0)
    pltpu.make_async_copy(zeros_hbm, bounce_v, asem).wait()
    plsc.subcore_barrier()

    @pl.when(s == 0)
    def _writeback():
        pltpu.sync_copy(deg_sh, bounce_v)
        pltpu.sync_copy(bounce_v, out_hbm.at[c])


@functools.partial(
    pl.kernel,
    out_type=jax.ShapeDtypeStruct((NC, N, F), jnp.float32),
    mesh=_mesh,
    compiler_params=_sc_params,
    scratch_types=[
        pltpu.VMEM((KCH, B), jnp.int32),
        pltpu.VMEM((KCH, B), jnp.int32),
        [pltpu.VMEM((B, F), jnp.float32)] * NBUF,
        pltpu.VMEM_SHARED((N, F), jnp.float32),
        [pltpu.SemaphoreType.DMA] * NBUF,
        [pltpu.SemaphoreType.DMA] * NBUF,
    ],
)
def _sc_scatter(ei_hbm, g_hbm, zeros_hbm, out_hbm,
                ridx_v, cidx_v, rows_v, acc_sh, gsem, ssem):
    c = lax.axis_index("c")
    s = lax.axis_index("s")
    w = c * NS + s

    pltpu.sync_copy(zeros_hbm, rows_v[0])
    pltpu.sync_copy(ei_hbm.at[0, pl.ds(w * KCH, KCH)], ridx_v)
    pltpu.sync_copy(ei_hbm.at[1, pl.ds(w * KCH, KCH)], cidx_v)
    for j in range(WB):
        pltpu.async_copy(rows_v[0], acc_sh.at[pl.ds(s * NPT + j * B, B)],
                         ssem[0])
    if WBR:
        pltpu.async_copy(rows_v[0].at[pl.ds(0, WBR)],
                         acc_sh.at[pl.ds(s * NPT + WB * B, WBR)], ssem[0])
    for j in range(WB):
        pltpu.make_async_copy(rows_v[0],
                              acc_sh.at[pl.ds(s * NPT + j * B, B)],
                              ssem[0]).wait()
    if WBR:
        pltpu.make_async_copy(rows_v[0].at[pl.ds(0, WBR)],
                              acc_sh.at[pl.ds(s * NPT + WB * B, WBR)],
                              ssem[0]).wait()
    plsc.subcore_barrier()

    for i in range(PF):
        pltpu.async_copy(g_hbm.at[ridx_v.at[i]], rows_v[i], gsem[i])

    def body(k0, carry):
        for phase in range(NBUF):
            k = k0 * NBUF + phase
            bi = phase % NBUF

            @pl.when(k < KCH)
            def _work():
                pltpu.make_async_copy(
                    g_hbm.at[ridx_v.at[k]], rows_v[bi], gsem[bi]).wait()
                pltpu.async_copy(rows_v[bi], acc_sh.at[cidx_v.at[k]],
                                 ssem[bi], add=True)

            bj = (phase + PF) % NBUF

            @pl.when(jnp.logical_and(k >= NBUF - PF, k + PF < KCH))
            def _drain_prev():
                pltpu.make_async_copy(
                    rows_v[bj], acc_sh.at[cidx_v.at[0]], ssem[bj]).wait()

            @pl.when(k + PF < KCH)
            def _prefetch():
                pltpu.async_copy(
                    g_hbm.at[ridx_v.at[k + PF]], rows_v[bj], gsem[bj])
        return carry

    lax.fori_loop(0, (KCH + NBUF - 1) // NBUF, body, 0)
    for bi in range(NBUF):
        pltpu.make_async_copy(rows_v[bi], acc_sh.at[cidx_v.at[0]],
                              ssem[bi]).wait()
    plsc.subcore_barrier()

    nwb = WB + (1 if WBR else 0)
    for j in range(nwb):
        bi = j % NBUF
        nrows = B if j < WB else WBR
        off = s * NPT + j * B
        src = rows_v[bi] if j < WB else rows_v[bi].at[pl.ds(0, WBR)]
        if j >= NBUF:
            poff = s * NPT + (j - NBUF) * B
            pltpu.make_async_copy(rows_v[bi], out_hbm.at[c, pl.ds(poff, B)],
                                  gsem[bi]).wait()
        pltpu.sync_copy(acc_sh.at[pl.ds(off, nrows)], src)
        pltpu.async_copy(src, out_hbm.at[c, pl.ds(off, nrows)], gsem[bi])
    for j in range(max(0, nwb - NBUF), nwb):
        bi = j % NBUF
        nrows = B if j < WB else WBR
        off = s * NPT + j * B
        src = rows_v[bi] if j < WB else rows_v[bi].at[pl.ds(0, WBR)]
        pltpu.make_async_copy(src, out_hbm.at[c, pl.ds(off, nrows)],
                              gsem[bi]).wait()


def _tc_linear_body(x_ref, w_ref, u_ref, h_ref):
    Wm = w_ref[...]
    uv = u_ref[...]
    v = jnp.dot(uv, Wm, preferred_element_type=jnp.float32)
    v = v / (jnp.sqrt(jnp.sum(v * v)) + 1e-12)
    wv = jnp.dot(v, Wm.T, preferred_element_type=jnp.float32)
    u2 = wv / (jnp.sqrt(jnp.sum(wv * wv)) + 1e-12)
    sigma = jnp.sum(u2 * wv)
    h_ref[...] = jnp.dot(x_ref[...], Wm,
                         preferred_element_type=jnp.float32) / sigma


def _tc_scale_body(h_ref, degp_ref, g_ref):
    deg = degp_ref[:, 0:1] + degp_ref[:, 1:2] + 1.0
    g_ref[...] = h_ref[...] * lax.rsqrt(deg)


def _tc_combine_body(acc_ref, g_ref, degp_ref, b_ref, a_ref, out_ref):
    total = acc_ref[0] + acc_ref[1] + g_ref[...]
    deg = degp_ref[:, 0:1] + degp_ref[:, 1:2] + 1.0
    o = total * lax.rsqrt(deg) + b_ref[...]
    a = a_ref[0, 0]
    out_ref[...] = jnp.where(o >= 0, o, a * o)


def kernel(x, edge_index, W, b, prelu_a, u):
    ei3 = edge_index.astype(jnp.int32).reshape(2, E // B, B)

    ones_b = jnp.ones((B,), jnp.float32)
    zeros_n = jnp.zeros((N,), jnp.float32)
    zeros_zr = jnp.zeros((B, F), jnp.float32)

    deg_p = _sc_degree(ei3, ones_b, zeros_n)
    degp_t = deg_p.T

    R = 2000
    grid = (N // R,)
    h = pl.pallas_call(
        _tc_linear_body,
        grid=grid,
        in_specs=[
            pl.BlockSpec((R, F), lambda i: (i, 0)),
            pl.BlockSpec((F, F), lambda i: (0, 0)),
            pl.BlockSpec((1, F), lambda i: (0, 0)),
        ],
        out_specs=pl.BlockSpec((R, F), lambda i: (i, 0)),
        out_shape=jax.ShapeDtypeStruct((N, F), jnp.float32),
    )(x, W, u.reshape(1, F))

    g = pl.pallas_call(
        _tc_scale_body,
        grid=grid,
        in_specs=[
            pl.BlockSpec((R, F), lambda i: (i, 0)),
            pl.BlockSpec((R, 2), lambda i: (i, 0)),
        ],
        out_specs=pl.BlockSpec((R, F), lambda i: (i, 0)),
        out_shape=jax.ShapeDtypeStruct((N, F), jnp.float32),
    )(h, degp_t)

    acc = _sc_scatter(ei3, g, zeros_zr)

    out = pl.pallas_call(
        _tc_combine_body,
        grid=grid,
        in_specs=[
            pl.BlockSpec((NC, R, F), lambda i: (0, i, 0)),
            pl.BlockSpec((R, F), lambda i: (i, 0)),
            pl.BlockSpec((R, 2), lambda i: (i, 0)),
            pl.BlockSpec((1, F), lambda i: (0, 0)),
            pl.BlockSpec((1, 1), lambda i: (0, 0)),
        ],
        out_specs=pl.BlockSpec((R, F), lambda i: (i, 0)),
        out_shape=jax.ShapeDtypeStruct((N, F), jnp.float32),
    )(acc, g, degp_t, b.reshape(1, F), prelu_a.reshape(1, 1))
    return out
